# baseline (speedup 1.0000x reference)
import functools

import jax
import jax.numpy as jnp
import numpy as np
from jax import lax
from jax.experimental import pallas as pl
from jax.experimental.pallas import tpu as pltpu
from jax.experimental.pallas import tpu_sc as plsc

N = 50000
E = 800000
H = 128
NB = 64
GW = 16
NGRP = H // GW
NC = 2
NS = 16
GPC = NGRP // NC

CH = 2048
EPT = 2 * CH * 13
CHUNKS = 13
E_PAD = NS * EPT
ACC_ROWS = N + 8
RPT = N // NS

RB = 1000
NBLK = N // RB

def _dot(a, b):
    return jax.lax.dot_general(a, b, (((1,), (0,)), ((), ())),
                               preferred_element_type=jnp.float32)


def _sc_body(hflat, srcg, dstp, m, htab, gidx0, gidx1, didx0, didx1,
             rows0, rows1, acc, sem_i, sem_g, sem_s):
    def i32c(x):
        return jnp.int32(x)

    c = lax.axis_index("c")
    s = lax.axis_index("s")
    e0 = s * i32c(EPT)
    r0 = s * i32c(RPT)

    for gi in range(GPC):
        g = c * i32c(GPC) + i32c(gi)
        pltpu.sync_copy(hflat.at[pl.ds(r0, RPT), pl.ds(g * i32c(GW), GW)],
                        htab.at[pl.ds(g * i32c(N) + r0, RPT)])
    plsc.subcore_barrier()

    for gi in range(GPC):
        g = c * i32c(GPC) + i32c(gi)
        gcol = g * i32c(GW)
        pltpu.sync_copy(hflat.at[pl.ds(r0, RPT), pl.ds(gcol, GW)],
                        acc.at[pl.ds(r0, RPT)])
        plsc.subcore_barrier()

        def chunk_body(p, carry):
            b0 = e0 + p * i32c(2 * CH)
            b1 = b0 + i32c(CH)
            ia0 = pltpu.async_copy(srcg.at[g, pl.ds(b0, CH)], gidx0, sem_i)
            ib0 = pltpu.async_copy(dstp.at[pl.ds(b0, CH)], didx0, sem_i)
            ia1 = pltpu.async_copy(srcg.at[g, pl.ds(b1, CH)], gidx1, sem_i)
            ib1 = pltpu.async_copy(dstp.at[pl.ds(b1, CH)], didx1, sem_i)
            ia0.wait()
            ib0.wait()
            dg0 = pltpu.async_copy(htab.at[gidx0], rows0, sem_g)
            ia1.wait()
            ib1.wait()
            dg1 = pltpu.async_copy(htab.at[gidx1], rows1, sem_g)
            dg0.wait()
            ds0 = pltpu.async_copy(rows0, acc.at[didx0], sem_s, add=True)
            dg1.wait()
            ds1 = pltpu.async_copy(rows1, acc.at[didx1], sem_s, add=True)
            ds0.wait()
            ds1.wait()
            return carry

        lax.fori_loop(jnp.int32(0), jnp.int32(CHUNKS), chunk_body, jnp.int32(0))
        plsc.subcore_barrier()
        pltpu.sync_copy(acc.at[pl.ds(r0, RPT)],
                        m.at[pl.ds(r0, RPT), pl.ds(gcol, GW)])
        plsc.subcore_barrier()


@functools.cache
def _build_sc_mp():
    return pl.kernel(
        _sc_body,
        out_type=[jax.ShapeDtypeStruct((N, H), jnp.float32),
                  jax.ShapeDtypeStruct((N * NGRP, GW), jnp.float32)],
        mesh=plsc.VectorSubcoreMesh(core_axis_name="c", subcore_axis_name="s",
                                    num_cores=NC, num_subcores=NS),
        compiler_params=pltpu.CompilerParams(use_tc_tiling_on_sc=False),
        scratch_types=[
            pltpu.VMEM((CH,), jnp.int32),
            pltpu.VMEM((CH,), jnp.int32),
            pltpu.VMEM((CH,), jnp.int32),
            pltpu.VMEM((CH,), jnp.int32),
            pltpu.VMEM((CH, GW), jnp.float32),
            pltpu.VMEM((CH, GW), jnp.float32),
            pltpu.VMEM_SHARED((ACC_ROWS, GW), jnp.float32),
            pltpu.SemaphoreType.DMA,
            pltpu.SemaphoreType.DMA,
            pltpu.SemaphoreType.DMA,
        ],
    )


def _inproj_body(x_ref, w_ref, b_ref, o_ref):
    o_ref[...] = jnp.maximum(_dot(x_ref[...], w_ref[...]) + b_ref[...], 0.0)


def _mlp_body(m_ref, w1_ref, b1_ref, w2_ref, b2_ref, o_ref):
    t = jnp.maximum(_dot(m_ref[...], w1_ref[...]) + b1_ref[...], 0.0)
    o_ref[...] = jnp.maximum(_dot(t, w2_ref[...]) + b2_ref[...], 0.0)


def _pool_body(h_ref, b_ref, sums_ref, cnt_ref):
    i = pl.program_id(0)
    ids = b_ref[0, 0, :]
    iot = lax.broadcasted_iota(jnp.int32, (NB, RB), 0)
    oh = (ids[None, :] == iot).astype(jnp.float32)
    ps = _dot(oh, h_ref[...])
    pc = jnp.broadcast_to(jnp.sum(oh, axis=1, keepdims=True), (NB, H))

    @pl.when(i == 0)
    def _():
        sums_ref[...] = jnp.zeros_like(sums_ref)
        cnt_ref[...] = jnp.zeros_like(cnt_ref)

    sums_ref[...] += ps
    cnt_ref[...] += pc


def _heads_body(sums_ref, cnt_ref, gf_ref, wfa_ref, wfb_ref, bf_ref,
                w1_ref, b1_ref, w2_ref, b2_ref, o_ref):
    pooled = sums_ref[...] / jnp.maximum(cnt_ref[...], 1.0)
    emb = jnp.maximum(_dot(pooled, wfa_ref[...]) + _dot(gf_ref[...], wfb_ref[...])
                      + bf_ref[...], 0.0)
    hid = jnp.maximum(_dot(emb, w1_ref[...]) + b1_ref[...], 0.0)
    o_ref[...] = _dot(hid, w2_ref[...]) + b2_ref[...]


_Z = np.int32(0)


def _row_blocked(cols):
    return pl.BlockSpec((RB, cols), lambda i: (i, _Z))


def _const(shape):
    nd = len(shape)
    return pl.BlockSpec(shape, lambda i: (_Z,) * nd)


_inproj = pl.pallas_call(
    _inproj_body,
    grid=(NBLK,),
    in_specs=[_row_blocked(8), _const((8, H)), _const((1, H))],
    out_specs=_row_blocked(H),
    out_shape=jax.ShapeDtypeStruct((N, H), jnp.float32),
)

_mlp = pl.pallas_call(
    _mlp_body,
    grid=(NBLK,),
    in_specs=[_row_blocked(H), _const((H, H)), _const((1, H)),
              _const((H, H)), _const((1, H))],
    out_specs=_row_blocked(H),
    out_shape=jax.ShapeDtypeStruct((N, H), jnp.float32),
)

_pool = pl.pallas_call(
    _pool_body,
    grid=(NBLK,),
    in_specs=[_row_blocked(H), pl.BlockSpec((1, 1, RB), lambda i: (i, _Z, _Z))],
    out_specs=[_const((NB, H)), _const((NB, H))],
    out_shape=[jax.ShapeDtypeStruct((NB, H), jnp.float32),
               jax.ShapeDtypeStruct((NB, H), jnp.float32)],
)

_heads = pl.pallas_call(
    _heads_body,
    out_shape=jax.ShapeDtypeStruct((NB, 16), jnp.float32),
)


def kernel(node_features, edge_index, global_features, batch,
           W_in, b_in,
           W1_0, b1_0, W2_0, b2_0,
           W1_1, b1_1, W2_1, b2_1,
           W1_2, b1_2, W2_2, b2_2,
           Wf, bf,
           Wd1, bd1, Wd2, bd2,
           Wr1, br1, Wr2, br2,
           Wv1, bv1, Wv2, bv2):
    f32 = jnp.float32
    i32 = jnp.int32
    nf = node_features.astype(f32)
    gf = global_features.astype(f32)

    src = edge_index[0].astype(i32)
    dst = edge_index[1].astype(i32)
    pad = E_PAD - E
    srcg = src[None, :] + (jnp.arange(NGRP, dtype=i32) * N)[:, None]
    srcg = jnp.concatenate([srcg, jnp.zeros((NGRP, pad), i32)], axis=1)
    dstp = jnp.concatenate([dst, jnp.full((pad,), N, i32)])
    batch3 = batch.astype(i32).reshape(NBLK, 1, RB)

    b_in2 = b_in.reshape(1, H).astype(f32)
    wfa = Wf[:H].astype(f32)
    wfb = Wf[H:].astype(f32)
    bf2 = bf.reshape(1, H).astype(f32)
    wh1 = jnp.concatenate([Wd1, Wr1, Wv1], axis=1).astype(f32)
    bh1 = jnp.concatenate([bd1, br1, bv1]).reshape(1, 3 * H).astype(f32)
    w2blk = jnp.zeros((3 * H, 16), f32)
    w2blk = w2blk.at[0:H, 0:6].set(Wd2.astype(f32))
    w2blk = w2blk.at[H:2 * H, 6:15].set(Wr2.astype(f32))
    w2blk = w2blk.at[2 * H:3 * H, 15:16].set(Wv2.astype(f32))
    b2blk = jnp.concatenate([bd2, br2, bv2]).reshape(1, 16).astype(f32)

    h = _inproj(nf, W_in.astype(f32), b_in2)
    layer_params = ((W1_0, b1_0, W2_0, b2_0),
                    (W1_1, b1_1, W2_1, b2_1),
                    (W1_2, b1_2, W2_2, b2_2))
    for (W1, b1, W2, b2) in layer_params:
        m, _ = _build_sc_mp()(h, srcg, dstp)
        h = _mlp(m, W1.astype(f32), b1.reshape(1, H).astype(f32),
                 W2.astype(f32), b2.reshape(1, H).astype(f32))

    sums, cnt = _pool(h, batch3)
    out16 = _heads(sums, cnt, gf, wfa, wfb, bf2, wh1, bh1, w2blk, b2blk)
    return out16[:, 0:6], out16[:, 6:15], out16[:, 15:16]


# speedup vs baseline: 2.9067x; 2.9067x over previous
import functools

import jax
import jax.numpy as jnp
import numpy as np
from jax import lax
from jax.experimental import pallas as pl
from jax.experimental.pallas import tpu as pltpu
from jax.experimental.pallas import tpu_sc as plsc

N = 50000
E = 800000
H = 128
NB = 64
GW = 32
NGRP = H // GW
NC = 2
NS = 16
GPC = NGRP // NC

CH = 448
CHUNKS = 56
EPT = 2 * CH * CHUNKS
E_PAD = NS * EPT
ACC_ROWS = N + 8
RPT = N // NS

RB = 1000
NBLK = N // RB

def _dot(a, b):
    return jax.lax.dot_general(a, b, (((1,), (0,)), ((), ())),
                               preferred_element_type=jnp.float32)


def _sc_body(hflat, srcg, dstp, m, htab, gidx0, gidx1, didx0, didx1,
             rows0, rows1, acc, sem_i, sem_g, sem_s):
    def i32c(x):
        return jnp.int32(x)

    c = lax.axis_index("c")
    s = lax.axis_index("s")
    e0 = s * i32c(EPT)
    r0 = s * i32c(RPT)

    for gi in range(GPC):
        g = c * i32c(GPC) + i32c(gi)
        pltpu.sync_copy(hflat.at[pl.ds(r0, RPT), pl.ds(g * i32c(GW), GW)],
                        htab.at[pl.ds(g * i32c(N) + r0, RPT)])
    plsc.subcore_barrier()

    for gi in range(GPC):
        g = c * i32c(GPC) + i32c(gi)
        gcol = g * i32c(GW)
        pltpu.sync_copy(hflat.at[pl.ds(r0, RPT), pl.ds(gcol, GW)],
                        acc.at[pl.ds(r0, RPT)])
        plsc.subcore_barrier()

        def chunk_body(p, carry):
            b0 = e0 + p * i32c(2 * CH)
            b1 = b0 + i32c(CH)
            ia0 = pltpu.async_copy(srcg.at[g, pl.ds(b0, CH)], gidx0, sem_i)
            ib0 = pltpu.async_copy(dstp.at[pl.ds(b0, CH)], didx0, sem_i)
            ia1 = pltpu.async_copy(srcg.at[g, pl.ds(b1, CH)], gidx1, sem_i)
            ib1 = pltpu.async_copy(dstp.at[pl.ds(b1, CH)], didx1, sem_i)
            ia0.wait()
            ib0.wait()
            dg0 = pltpu.async_copy(htab.at[gidx0], rows0, sem_g)
            ia1.wait()
            ib1.wait()
            dg1 = pltpu.async_copy(htab.at[gidx1], rows1, sem_g)
            dg0.wait()
            ds0 = pltpu.async_copy(rows0, acc.at[didx0], sem_s, add=True)
            dg1.wait()
            ds1 = pltpu.async_copy(rows1, acc.at[didx1], sem_s, add=True)
            ds0.wait()
            ds1.wait()
            return carry

        lax.fori_loop(jnp.int32(0), jnp.int32(CHUNKS), chunk_body, jnp.int32(0))
        plsc.subcore_barrier()
        pltpu.sync_copy(acc.at[pl.ds(r0, RPT)],
                        m.at[pl.ds(r0, RPT), pl.ds(gcol, GW)])
        plsc.subcore_barrier()


@functools.cache
def _build_sc_mp():
    return pl.kernel(
        _sc_body,
        out_type=[jax.ShapeDtypeStruct((N, H), jnp.float32),
                  jax.ShapeDtypeStruct((N * NGRP, GW), jnp.float32)],
        mesh=plsc.VectorSubcoreMesh(core_axis_name="c", subcore_axis_name="s",
                                    num_cores=NC, num_subcores=NS),
        compiler_params=pltpu.CompilerParams(use_tc_tiling_on_sc=False),
        scratch_types=[
            pltpu.VMEM((CH,), jnp.int32),
            pltpu.VMEM((CH,), jnp.int32),
            pltpu.VMEM((CH,), jnp.int32),
            pltpu.VMEM((CH,), jnp.int32),
            pltpu.VMEM((CH, GW), jnp.float32),
            pltpu.VMEM((CH, GW), jnp.float32),
            pltpu.VMEM_SHARED((ACC_ROWS, GW), jnp.float32),
            pltpu.SemaphoreType.DMA,
            pltpu.SemaphoreType.DMA,
            pltpu.SemaphoreType.DMA,
        ],
    )


def _inproj_body(x_ref, w_ref, b_ref, o_ref):
    o_ref[...] = jnp.maximum(_dot(x_ref[...], w_ref[...]) + b_ref[...], 0.0)


def _mlp_body(m_ref, w1_ref, b1_ref, w2_ref, b2_ref, o_ref):
    t = jnp.maximum(_dot(m_ref[...], w1_ref[...]) + b1_ref[...], 0.0)
    o_ref[...] = jnp.maximum(_dot(t, w2_ref[...]) + b2_ref[...], 0.0)


def _pool_body(h_ref, b_ref, sums_ref, cnt_ref):
    i = pl.program_id(0)
    ids = b_ref[0, 0, :]
    iot = lax.broadcasted_iota(jnp.int32, (NB, RB), 0)
    oh = (ids[None, :] == iot).astype(jnp.float32)
    ps = _dot(oh, h_ref[...])
    pc = jnp.broadcast_to(jnp.sum(oh, axis=1, keepdims=True), (NB, H))

    @pl.when(i == 0)
    def _():
        sums_ref[...] = jnp.zeros_like(sums_ref)
        cnt_ref[...] = jnp.zeros_like(cnt_ref)

    sums_ref[...] += ps
    cnt_ref[...] += pc


def _heads_body(sums_ref, cnt_ref, gf_ref, wfa_ref, wfb_ref, bf_ref,
                w1_ref, b1_ref, w2_ref, b2_ref, o_ref):
    pooled = sums_ref[...] / jnp.maximum(cnt_ref[...], 1.0)
    emb = jnp.maximum(_dot(pooled, wfa_ref[...]) + _dot(gf_ref[...], wfb_ref[...])
                      + bf_ref[...], 0.0)
    hid = jnp.maximum(_dot(emb, w1_ref[...]) + b1_ref[...], 0.0)
    o_ref[...] = _dot(hid, w2_ref[...]) + b2_ref[...]


_Z = np.int32(0)


def _row_blocked(cols):
    return pl.BlockSpec((RB, cols), lambda i: (i, _Z))


def _const(shape):
    nd = len(shape)
    return pl.BlockSpec(shape, lambda i: (_Z,) * nd)


_inproj = pl.pallas_call(
    _inproj_body,
    grid=(NBLK,),
    in_specs=[_row_blocked(8), _const((8, H)), _const((1, H))],
    out_specs=_row_blocked(H),
    out_shape=jax.ShapeDtypeStruct((N, H), jnp.float32),
)

_mlp = pl.pallas_call(
    _mlp_body,
    grid=(NBLK,),
    in_specs=[_row_blocked(H), _const((H, H)), _const((1, H)),
              _const((H, H)), _const((1, H))],
    out_specs=_row_blocked(H),
    out_shape=jax.ShapeDtypeStruct((N, H), jnp.float32),
)

_pool = pl.pallas_call(
    _pool_body,
    grid=(NBLK,),
    in_specs=[_row_blocked(H), pl.BlockSpec((1, 1, RB), lambda i: (i, _Z, _Z))],
    out_specs=[_const((NB, H)), _const((NB, H))],
    out_shape=[jax.ShapeDtypeStruct((NB, H), jnp.float32),
               jax.ShapeDtypeStruct((NB, H), jnp.float32)],
)

_heads = pl.pallas_call(
    _heads_body,
    out_shape=jax.ShapeDtypeStruct((NB, 16), jnp.float32),
)


def kernel(node_features, edge_index, global_features, batch,
           W_in, b_in,
           W1_0, b1_0, W2_0, b2_0,
           W1_1, b1_1, W2_1, b2_1,
           W1_2, b1_2, W2_2, b2_2,
           Wf, bf,
           Wd1, bd1, Wd2, bd2,
           Wr1, br1, Wr2, br2,
           Wv1, bv1, Wv2, bv2):
    f32 = jnp.float32
    i32 = jnp.int32
    nf = node_features.astype(f32)
    gf = global_features.astype(f32)

    src = edge_index[0].astype(i32)
    dst = edge_index[1].astype(i32)
    pad = E_PAD - E
    srcg = src[None, :] + (jnp.arange(NGRP, dtype=i32) * N)[:, None]
    srcg = jnp.concatenate([srcg, jnp.zeros((NGRP, pad), i32)], axis=1)
    dstp = jnp.concatenate([dst, jnp.full((pad,), N, i32)])
    batch3 = batch.astype(i32).reshape(NBLK, 1, RB)

    b_in2 = b_in.reshape(1, H).astype(f32)
    wfa = Wf[:H].astype(f32)
    wfb = Wf[H:].astype(f32)
    bf2 = bf.reshape(1, H).astype(f32)
    wh1 = jnp.concatenate([Wd1, Wr1, Wv1], axis=1).astype(f32)
    bh1 = jnp.concatenate([bd1, br1, bv1]).reshape(1, 3 * H).astype(f32)
    w2blk = jnp.zeros((3 * H, 16), f32)
    w2blk = w2blk.at[0:H, 0:6].set(Wd2.astype(f32))
    w2blk = w2blk.at[H:2 * H, 6:15].set(Wr2.astype(f32))
    w2blk = w2blk.at[2 * H:3 * H, 15:16].set(Wv2.astype(f32))
    b2blk = jnp.concatenate([bd2, br2, bv2]).reshape(1, 16).astype(f32)

    h = _inproj(nf, W_in.astype(f32), b_in2)
    layer_params = ((W1_0, b1_0, W2_0, b2_0),
                    (W1_1, b1_1, W2_1, b2_1),
                    (W1_2, b1_2, W2_2, b2_2))
    for (W1, b1, W2, b2) in layer_params:
        m, _ = _build_sc_mp()(h, srcg, dstp)
        h = _mlp(m, W1.astype(f32), b1.reshape(1, H).astype(f32),
                 W2.astype(f32), b2.reshape(1, H).astype(f32))

    sums, cnt = _pool(h, batch3)
    out16 = _heads(sums, cnt, gf, wfa, wfb, bf2, wh1, bh1, w2blk, b2blk)
    return out16[:, 0:6], out16[:, 6:15], out16[:, 15:16]


# speedup vs baseline: 5.1260x; 1.7635x over previous
import functools

import jax
import jax.numpy as jnp
import numpy as np
from jax import lax
from jax.experimental import pallas as pl
from jax.experimental.pallas import tpu as pltpu
from jax.experimental.pallas import tpu_sc as plsc

N = 50000
E = 800000
H = 128
NB = 64
GW = 32
NGRP = H // GW
NC = 2
NS = 16
GPC = NGRP // NC

CH = 448
CHUNKS = 56
EPT = 2 * CH * CHUNKS
E_PAD = NS * EPT
ACC_ROWS = N + 8
RPT = N // NS

RB = 1000
NBLK = N // RB

def _dot(a, b):
    return jax.lax.dot_general(a, b, (((1,), (0,)), ((), ())),
                               preferred_element_type=jnp.float32)


def _sc_body(htab, srcg, dstp, m, gidx0, gidx1, didx0, didx1,
             rows0, rows1, acc, sem_i, sem_g, sem_s):
    def i32c(x):
        return jnp.int32(x)

    c = lax.axis_index("c")
    s = lax.axis_index("s")
    e0 = s * i32c(EPT)
    r0 = s * i32c(RPT)

    for gi in range(GPC):
        g = c * i32c(GPC) + i32c(gi)
        gcol = g * i32c(GW)
        pltpu.sync_copy(htab.at[pl.ds(g * i32c(N) + r0, RPT)],
                        acc.at[pl.ds(r0, RPT)])
        plsc.subcore_barrier()

        def chunk_body(p, carry):
            b0 = e0 + p * i32c(2 * CH)
            b1 = b0 + i32c(CH)
            ia0 = pltpu.async_copy(srcg.at[g, pl.ds(b0, CH)], gidx0, sem_i)
            ib0 = pltpu.async_copy(dstp.at[pl.ds(b0, CH)], didx0, sem_i)
            ia1 = pltpu.async_copy(srcg.at[g, pl.ds(b1, CH)], gidx1, sem_i)
            ib1 = pltpu.async_copy(dstp.at[pl.ds(b1, CH)], didx1, sem_i)
            ia0.wait()
            ib0.wait()
            dg0 = pltpu.async_copy(htab.at[gidx0], rows0, sem_g)
            ia1.wait()
            ib1.wait()
            dg1 = pltpu.async_copy(htab.at[gidx1], rows1, sem_g)
            dg0.wait()
            ds0 = pltpu.async_copy(rows0, acc.at[didx0], sem_s, add=True)
            dg1.wait()
            ds1 = pltpu.async_copy(rows1, acc.at[didx1], sem_s, add=True)
            ds0.wait()
            ds1.wait()
            return carry

        lax.fori_loop(jnp.int32(0), jnp.int32(CHUNKS), chunk_body, jnp.int32(0))
        plsc.subcore_barrier()
        pltpu.sync_copy(acc.at[pl.ds(r0, RPT)],
                        m.at[pl.ds(r0, RPT), pl.ds(gcol, GW)])
        plsc.subcore_barrier()


@functools.cache
def _build_sc_mp():
    return pl.kernel(
        _sc_body,
        out_type=jax.ShapeDtypeStruct((N, H), jnp.float32),
        mesh=plsc.VectorSubcoreMesh(core_axis_name="c", subcore_axis_name="s",
                                    num_cores=NC, num_subcores=NS),
        compiler_params=pltpu.CompilerParams(use_tc_tiling_on_sc=False),
        scratch_types=[
            pltpu.VMEM((CH,), jnp.int32),
            pltpu.VMEM((CH,), jnp.int32),
            pltpu.VMEM((CH,), jnp.int32),
            pltpu.VMEM((CH,), jnp.int32),
            pltpu.VMEM((CH, GW), jnp.float32),
            pltpu.VMEM((CH, GW), jnp.float32),
            pltpu.VMEM_SHARED((ACC_ROWS, GW), jnp.float32),
            pltpu.SemaphoreType.DMA,
            pltpu.SemaphoreType.DMA,
            pltpu.SemaphoreType.DMA,
        ],
    )


def _emit_slab(y_when_g0, oslab_ref, scr_ref):
    gdim = pl.program_id(1)

    @pl.when(gdim == 0)
    def _():
        scr_ref[...] = y_when_g0()

    for g in range(NGRP):
        @pl.when(gdim == g)
        def _():
            oslab_ref[...] = scr_ref[:, g * GW:(g + 1) * GW]


def _inproj_body(x_ref, w_ref, b_ref, oslab_ref, scr_ref):
    _emit_slab(
        lambda: jnp.maximum(_dot(x_ref[...], w_ref[...]) + b_ref[...], 0.0),
        oslab_ref, scr_ref)


def _mlp_slab_body(m_ref, w1_ref, b1_ref, w2_ref, b2_ref, oslab_ref, scr_ref):
    def compute():
        t = jnp.maximum(_dot(m_ref[...], w1_ref[...]) + b1_ref[...], 0.0)
        return jnp.maximum(_dot(t, w2_ref[...]) + b2_ref[...], 0.0)

    _emit_slab(compute, oslab_ref, scr_ref)


def _mlp_body(m_ref, w1_ref, b1_ref, w2_ref, b2_ref, o_ref):
    t = jnp.maximum(_dot(m_ref[...], w1_ref[...]) + b1_ref[...], 0.0)
    o_ref[...] = jnp.maximum(_dot(t, w2_ref[...]) + b2_ref[...], 0.0)


def _pool_body(h_ref, b_ref, sums_ref, cnt_ref):
    i = pl.program_id(0)
    ids = b_ref[0, 0, :]
    iot = lax.broadcasted_iota(jnp.int32, (NB, RB), 0)
    oh = (ids[None, :] == iot).astype(jnp.float32)
    ps = _dot(oh, h_ref[...])
    pc = jnp.broadcast_to(jnp.sum(oh, axis=1, keepdims=True), (NB, H))

    @pl.when(i == 0)
    def _():
        sums_ref[...] = jnp.zeros_like(sums_ref)
        cnt_ref[...] = jnp.zeros_like(cnt_ref)

    sums_ref[...] += ps
    cnt_ref[...] += pc


def _heads_body(sums_ref, cnt_ref, gf_ref, wfa_ref, wfb_ref, bf_ref,
                w1_ref, b1_ref, w2_ref, b2_ref, o_ref):
    pooled = sums_ref[...] / jnp.maximum(cnt_ref[...], 1.0)
    emb = jnp.maximum(_dot(pooled, wfa_ref[...]) + _dot(gf_ref[...], wfb_ref[...])
                      + bf_ref[...], 0.0)
    hid = jnp.maximum(_dot(emb, w1_ref[...]) + b1_ref[...], 0.0)
    o_ref[...] = _dot(hid, w2_ref[...]) + b2_ref[...]


_Z = np.int32(0)


def _row_blocked(cols):
    return pl.BlockSpec((RB, cols), lambda i: (i, _Z))


def _const(shape):
    nd = len(shape)
    return pl.BlockSpec(shape, lambda i: (_Z,) * nd)


def _row_blocked2(cols):
    return pl.BlockSpec((RB, cols), lambda i, g: (i, _Z))


def _const2(shape):
    nd = len(shape)
    return pl.BlockSpec(shape, lambda i, g: (_Z,) * nd)


_NBLK32 = np.int32(NBLK)
_slab_spec = pl.BlockSpec((RB, GW), lambda i, g: (g * _NBLK32 + i, _Z))
_slab_shape = jax.ShapeDtypeStruct((NGRP * N, GW), jnp.float32)

_inproj = pl.pallas_call(
    _inproj_body,
    grid=(NBLK, NGRP),
    in_specs=[_row_blocked2(8), _const2((8, H)), _const2((1, H))],
    out_specs=_slab_spec,
    out_shape=_slab_shape,
    scratch_shapes=[pltpu.VMEM((RB, H), jnp.float32)],
)

_mlp_slab = pl.pallas_call(
    _mlp_slab_body,
    grid=(NBLK, NGRP),
    in_specs=[_row_blocked2(H), _const2((H, H)), _const2((1, H)),
              _const2((H, H)), _const2((1, H))],
    out_specs=_slab_spec,
    out_shape=_slab_shape,
    scratch_shapes=[pltpu.VMEM((RB, H), jnp.float32)],
)

_mlp = pl.pallas_call(
    _mlp_body,
    grid=(NBLK,),
    in_specs=[_row_blocked(H), _const((H, H)), _const((1, H)),
              _const((H, H)), _const((1, H))],
    out_specs=_row_blocked(H),
    out_shape=jax.ShapeDtypeStruct((N, H), jnp.float32),
)

_pool = pl.pallas_call(
    _pool_body,
    grid=(NBLK,),
    in_specs=[_row_blocked(H), pl.BlockSpec((1, 1, RB), lambda i: (i, _Z, _Z))],
    out_specs=[_const((NB, H)), _const((NB, H))],
    out_shape=[jax.ShapeDtypeStruct((NB, H), jnp.float32),
               jax.ShapeDtypeStruct((NB, H), jnp.float32)],
)

_heads = pl.pallas_call(
    _heads_body,
    out_shape=jax.ShapeDtypeStruct((NB, 16), jnp.float32),
)


def kernel(node_features, edge_index, global_features, batch,
           W_in, b_in,
           W1_0, b1_0, W2_0, b2_0,
           W1_1, b1_1, W2_1, b2_1,
           W1_2, b1_2, W2_2, b2_2,
           Wf, bf,
           Wd1, bd1, Wd2, bd2,
           Wr1, br1, Wr2, br2,
           Wv1, bv1, Wv2, bv2):
    f32 = jnp.float32
    i32 = jnp.int32
    nf = node_features.astype(f32)
    gf = global_features.astype(f32)

    src = edge_index[0].astype(i32)
    dst = edge_index[1].astype(i32)
    pad = E_PAD - E
    srcg = src[None, :] + (jnp.arange(NGRP, dtype=i32) * N)[:, None]
    srcg = jnp.concatenate([srcg, jnp.zeros((NGRP, pad), i32)], axis=1)
    dstp = jnp.concatenate([dst, jnp.full((pad,), N, i32)])
    batch3 = batch.astype(i32).reshape(NBLK, 1, RB)

    b_in2 = b_in.reshape(1, H).astype(f32)
    wfa = Wf[:H].astype(f32)
    wfb = Wf[H:].astype(f32)
    bf2 = bf.reshape(1, H).astype(f32)
    wh1 = jnp.concatenate([Wd1, Wr1, Wv1], axis=1).astype(f32)
    bh1 = jnp.concatenate([bd1, br1, bv1]).reshape(1, 3 * H).astype(f32)
    w2blk = jnp.zeros((3 * H, 16), f32)
    w2blk = w2blk.at[0:H, 0:6].set(Wd2.astype(f32))
    w2blk = w2blk.at[H:2 * H, 6:15].set(Wr2.astype(f32))
    w2blk = w2blk.at[2 * H:3 * H, 15:16].set(Wv2.astype(f32))
    b2blk = jnp.concatenate([bd2, br2, bv2]).reshape(1, 16).astype(f32)

    htab = _inproj(nf, W_in.astype(f32), b_in2)
    layer_params = ((W1_0, b1_0, W2_0, b2_0),
                    (W1_1, b1_1, W2_1, b2_1),
                    (W1_2, b1_2, W2_2, b2_2))
    for l, (W1, b1, W2, b2) in enumerate(layer_params):
        m = _build_sc_mp()(htab, srcg, dstp)
        args = (m, W1.astype(f32), b1.reshape(1, H).astype(f32),
                W2.astype(f32), b2.reshape(1, H).astype(f32))
        if l < 2:
            htab = _mlp_slab(*args)
        else:
            h = _mlp(*args)

    sums, cnt = _pool(h, batch3)
    out16 = _heads(sums, cnt, gf, wfa, wfb, bf2, wh1, bh1, w2blk, b2blk)
    return out16[:, 0:6], out16[:, 6:15], out16[:, 15:16]
